# Initial kernel scaffold; baseline (speedup 1.0000x reference)
import jax
import jax.numpy as jnp
from jax.experimental import pallas as pl


def kernel(z, codebooks):
    raise NotImplementedError("write your pallas kernel here")


# speedup vs baseline: 1.9713x; 1.9713x over previous
import functools

import jax
import jax.numpy as jnp
from jax.experimental import pallas as pl

_NUM_PQ = 4
_NUM_CODES = 1024
_EMBED_DIM = 384
_PQ_DIM = _EMBED_DIM // _NUM_PQ
_ROWS = 16 * 24 * 24
_BLK_R = 1024
_N_RB = _ROWS // _BLK_R


def _vq_block(z_ref, cb_ref, zq_ref, prob_ref, loss_ref):
    g = pl.program_id(0)
    r = pl.program_id(1)

    zb = z_ref[0]
    cb = cb_ref[0]

    zsq = jnp.sum(zb * zb, axis=1, keepdims=True)
    csq = jnp.sum(cb * cb, axis=1, keepdims=True).T
    cross = jax.lax.dot_general(
        zb, cb, (((1,), (1,)), ((), ())),
        preferred_element_type=jnp.float32)
    dmat = (zsq + csq) - 2.0 * cross

    neg = dmat * -2.0
    m = jnp.max(neg, axis=1, keepdims=True)
    e = jnp.exp(neg - m)
    s = jnp.sum(e, axis=1, keepdims=True)
    prob_ref[...] = e / s

    dmin = jnp.min(dmat, axis=1, keepdims=True)
    iota = jax.lax.broadcasted_iota(jnp.int32, dmat.shape, 1)
    midx = jnp.min(jnp.where(dmat == dmin, iota, _NUM_CODES), axis=1)

    oh = (iota == midx[:, None]).astype(jnp.float32)
    zq = jax.lax.dot_general(
        oh, cb, (((1,), (0,)), ((), ())),
        preferred_element_type=jnp.float32,
        precision=jax.lax.Precision.HIGHEST)

    zq_ref[0] = zb + (zq - zb)

    part = jnp.sum((zq - zb) * (zq - zb)).reshape(1, 1)
    is_first = jnp.logical_and(g == 0, r == 0)
    prev = jnp.where(is_first, jnp.zeros((1, 1), jnp.float32), loss_ref[...])
    loss_ref[...] = prev + part


@jax.jit
def kernel(z, codebooks):
    B, C, H, W = z.shape
    z_g = jnp.transpose(
        z.reshape(B, _NUM_PQ, _PQ_DIM, H, W), (1, 0, 3, 4, 2)
    ).reshape(_NUM_PQ, _ROWS, _PQ_DIM)

    zq_g, prob, loss = pl.pallas_call(
        _vq_block,
        grid=(_NUM_PQ, _N_RB),
        in_specs=[
            pl.BlockSpec((1, _BLK_R, _PQ_DIM), lambda g, r: (g, r, 0)),
            pl.BlockSpec((1, _NUM_CODES, _PQ_DIM), lambda g, r: (g, 0, 0)),
        ],
        out_specs=[
            pl.BlockSpec((1, _BLK_R, _PQ_DIM), lambda g, r: (g, r, 0)),
            pl.BlockSpec((_BLK_R, _NUM_CODES), lambda g, r: (r, g)),
            pl.BlockSpec((1, 1), lambda g, r: (0, 0)),
        ],
        out_shape=[
            jax.ShapeDtypeStruct((_NUM_PQ, _ROWS, _PQ_DIM), jnp.float32),
            jax.ShapeDtypeStruct((_ROWS, _NUM_PQ * _NUM_CODES), jnp.float32),
            jax.ShapeDtypeStruct((1, 1), jnp.float32),
        ],
    )(z_g, codebooks)

    z_q = jnp.transpose(
        zq_g.reshape(_NUM_PQ, B, H, W, _PQ_DIM), (1, 0, 4, 2, 3)
    ).reshape(B, C, H, W)
    vq_loss = loss[0, 0] * (1.25 / (_NUM_PQ * _ROWS * _PQ_DIM))
    return z_q, vq_loss, prob


# speedup vs baseline: 2.8520x; 1.4467x over previous
import functools

import jax
import jax.numpy as jnp
from jax.experimental import pallas as pl

_NUM_PQ = 4
_NUM_CODES = 1024
_EMBED_DIM = 384
_PQ_DIM = _EMBED_DIM // _NUM_PQ
_ROWS = 16 * 24 * 24
_BLK_R = 1024
_N_RB = _ROWS // _BLK_R


def _vq_block(z_ref, cb_ref, zq_ref, prob_ref, loss_ref):
    g = pl.program_id(0)
    r = pl.program_id(1)

    zb = z_ref[0]
    cb = cb_ref[0]

    zsq = jnp.sum(zb * zb, axis=1, keepdims=True)
    csq = jnp.sum(cb * cb, axis=1, keepdims=True).T
    cross = jax.lax.dot_general(
        zb, cb, (((1,), (1,)), ((), ())),
        preferred_element_type=jnp.float32)
    dmat = (zsq + csq) - 2.0 * cross

    dmin = jnp.min(dmat, axis=1, keepdims=True)

    e = jnp.exp((dmat - dmin) * -2.0)
    s = jnp.sum(e, axis=1, keepdims=True)
    prob_ref[...] = e / s

    iota = jax.lax.broadcasted_iota(jnp.int32, dmat.shape, 1)
    midx = jnp.min(jnp.where(dmat == dmin, iota, _NUM_CODES), axis=1)

    oh = jnp.where(iota == midx[:, None], 1.0, 0.0).astype(jnp.bfloat16)
    zq = jax.lax.dot_general(
        oh, cb.astype(jnp.bfloat16), (((1,), (0,)), ((), ())),
        preferred_element_type=jnp.float32)

    zq_ref[0] = zb + (zq - zb)

    part = jnp.sum(dmin).reshape(1, 1)
    is_first = jnp.logical_and(g == 0, r == 0)
    prev = jnp.where(is_first, jnp.zeros((1, 1), jnp.float32), loss_ref[...])
    loss_ref[...] = prev + part


@jax.jit
def kernel(z, codebooks):
    B, C, H, W = z.shape
    z_g = jnp.transpose(
        z.reshape(B, _NUM_PQ, _PQ_DIM, H, W), (1, 0, 3, 4, 2)
    ).reshape(_NUM_PQ, _ROWS, _PQ_DIM)

    zq_g, prob, loss = pl.pallas_call(
        _vq_block,
        grid=(_NUM_PQ, _N_RB),
        in_specs=[
            pl.BlockSpec((1, _BLK_R, _PQ_DIM), lambda g, r: (g, r, 0)),
            pl.BlockSpec((1, _NUM_CODES, _PQ_DIM), lambda g, r: (g, 0, 0)),
        ],
        out_specs=[
            pl.BlockSpec((1, _BLK_R, _PQ_DIM), lambda g, r: (g, r, 0)),
            pl.BlockSpec((_BLK_R, _NUM_CODES), lambda g, r: (r, g)),
            pl.BlockSpec((1, 1), lambda g, r: (0, 0)),
        ],
        out_shape=[
            jax.ShapeDtypeStruct((_NUM_PQ, _ROWS, _PQ_DIM), jnp.float32),
            jax.ShapeDtypeStruct((_ROWS, _NUM_PQ * _NUM_CODES), jnp.float32),
            jax.ShapeDtypeStruct((1, 1), jnp.float32),
        ],
    )(z_g, codebooks)

    z_q = jnp.transpose(
        zq_g.reshape(_NUM_PQ, B, H, W, _PQ_DIM), (1, 0, 4, 2, 3)
    ).reshape(B, C, H, W)
    vq_loss = loss[0, 0] * (1.25 / (_NUM_PQ * _ROWS * _PQ_DIM))
    return z_q, vq_loss, prob


# speedup vs baseline: 2.9260x; 1.0260x over previous
import functools

import jax
import jax.numpy as jnp
from jax.experimental import pallas as pl
from jax.experimental.pallas import tpu as pltpu

_NUM_PQ = 4
_NUM_CODES = 1024
_EMBED_DIM = 384
_PQ_DIM = _EMBED_DIM // _NUM_PQ
_ROWS = 16 * 24 * 24
_BLK_R = 1024
_N_RB = _ROWS // _BLK_R


def _vq_block(z_ref, cb_ref, zq_ref, prob_ref, loss_ref):
    g = pl.program_id(0)
    r = pl.program_id(1)

    zb = z_ref[0]
    cb = cb_ref[0]

    zsq = jnp.sum(zb * zb, axis=1, keepdims=True)
    csq = jnp.sum(cb * cb, axis=1, keepdims=True).T
    cross = jax.lax.dot_general(
        zb, cb, (((1,), (1,)), ((), ())),
        preferred_element_type=jnp.float32)
    dmat = (zsq + csq) - 2.0 * cross

    dmin = jnp.min(dmat, axis=1, keepdims=True)
    t = dmat - dmin

    e = jnp.exp(t * -2.0)
    s = jnp.sum(e, axis=1, keepdims=True)
    prob_ref[...] = e * (1.0 / s)

    iota_f = jax.lax.broadcasted_iota(
        jnp.int32, dmat.shape, 1).astype(jnp.float32)
    key = t * jnp.float32(2.0 ** 34) + iota_f
    kmin = jnp.min(key, axis=1, keepdims=True)

    oh = jnp.where(key == kmin, 1.0, 0.0).astype(jnp.bfloat16)
    zq = jax.lax.dot_general(
        oh, cb.astype(jnp.bfloat16), (((1,), (0,)), ((), ())),
        preferred_element_type=jnp.float32)

    zq_ref[0] = zb + (zq - zb)

    loss_ref[0, 0, 0, 0] = jnp.sum(dmin)


@jax.jit
def kernel(z, codebooks):
    B, C, H, W = z.shape
    z_g = jnp.transpose(
        z.reshape(B, _NUM_PQ, _PQ_DIM, H, W), (1, 0, 3, 4, 2)
    ).reshape(_NUM_PQ, _ROWS, _PQ_DIM)

    zq_g, prob, loss = pl.pallas_call(
        _vq_block,
        grid=(_NUM_PQ, _N_RB),
        in_specs=[
            pl.BlockSpec((1, _BLK_R, _PQ_DIM), lambda g, r: (g, r, 0)),
            pl.BlockSpec((1, _NUM_CODES, _PQ_DIM), lambda g, r: (g, 0, 0)),
        ],
        out_specs=[
            pl.BlockSpec((1, _BLK_R, _PQ_DIM), lambda g, r: (g, r, 0)),
            pl.BlockSpec((_BLK_R, _NUM_CODES), lambda g, r: (r, g)),
            pl.BlockSpec((1, 1, 1, 1), lambda g, r: (g, r, 0, 0),
                         memory_space=pltpu.MemorySpace.SMEM),
        ],
        out_shape=[
            jax.ShapeDtypeStruct((_NUM_PQ, _ROWS, _PQ_DIM), jnp.float32),
            jax.ShapeDtypeStruct((_ROWS, _NUM_PQ * _NUM_CODES), jnp.float32),
            jax.ShapeDtypeStruct((_NUM_PQ, _N_RB, 1, 1), jnp.float32),
        ],
        compiler_params=pltpu.CompilerParams(
            dimension_semantics=("parallel", "parallel")),
    )(z_g, codebooks)

    z_q = jnp.transpose(
        zq_g.reshape(_NUM_PQ, B, H, W, _PQ_DIM), (1, 0, 4, 2, 3)
    ).reshape(B, C, H, W)
    vq_loss = jnp.sum(loss) * (1.25 / (_NUM_PQ * _ROWS * _PQ_DIM))
    return z_q, vq_loss, prob


# speedup vs baseline: 2.9722x; 1.0158x over previous
import functools

import jax
import jax.numpy as jnp
from jax.experimental import pallas as pl
from jax.experimental.pallas import tpu as pltpu

_NUM_PQ = 4
_NUM_CODES = 1024
_EMBED_DIM = 384
_PQ_DIM = _EMBED_DIM // _NUM_PQ
_ROWS = 16 * 24 * 24
_BLK_R = 1536
_N_RB = _ROWS // _BLK_R


def _vq_block(z_ref, cb_ref, zq_ref, prob_ref, loss_ref):
    g = pl.program_id(0)
    r = pl.program_id(1)

    zb = z_ref[0]
    cb = cb_ref[0]

    zsq = jnp.sum(zb * zb, axis=1, keepdims=True)
    csq = jnp.sum(cb * cb, axis=1, keepdims=True).T
    cross = jax.lax.dot_general(
        zb, cb, (((1,), (1,)), ((), ())),
        preferred_element_type=jnp.float32)
    dmat = (zsq + csq) - 2.0 * cross

    dmin = jnp.min(dmat, axis=1, keepdims=True)
    t = dmat - dmin

    e = jnp.exp(t * -2.0)
    s = jnp.sum(e, axis=1, keepdims=True)
    prob_ref[...] = e * (1.0 / s)

    iota_f = jax.lax.broadcasted_iota(
        jnp.int32, dmat.shape, 1).astype(jnp.float32)
    key = t * jnp.float32(2.0 ** 34) + iota_f
    kmin = jnp.min(key, axis=1, keepdims=True)

    oh = jnp.where(key == kmin, 1.0, 0.0).astype(jnp.bfloat16)
    zq = jax.lax.dot_general(
        oh, cb.astype(jnp.bfloat16), (((1,), (0,)), ((), ())),
        preferred_element_type=jnp.float32)

    zq_ref[0] = zb + (zq - zb)

    loss_ref[0, 0, 0, 0] = jnp.sum(dmin)


@jax.jit
def kernel(z, codebooks):
    B, C, H, W = z.shape
    z_g = jnp.transpose(
        z.reshape(B, _NUM_PQ, _PQ_DIM, H, W), (1, 0, 3, 4, 2)
    ).reshape(_NUM_PQ, _ROWS, _PQ_DIM)

    zq_g, prob, loss = pl.pallas_call(
        _vq_block,
        grid=(_NUM_PQ, _N_RB),
        in_specs=[
            pl.BlockSpec((1, _BLK_R, _PQ_DIM), lambda g, r: (g, r, 0)),
            pl.BlockSpec((1, _NUM_CODES, _PQ_DIM), lambda g, r: (g, 0, 0)),
        ],
        out_specs=[
            pl.BlockSpec((1, _BLK_R, _PQ_DIM), lambda g, r: (g, r, 0)),
            pl.BlockSpec((_BLK_R, _NUM_CODES), lambda g, r: (r, g)),
            pl.BlockSpec((1, 1, 1, 1), lambda g, r: (g, r, 0, 0),
                         memory_space=pltpu.MemorySpace.SMEM),
        ],
        out_shape=[
            jax.ShapeDtypeStruct((_NUM_PQ, _ROWS, _PQ_DIM), jnp.float32),
            jax.ShapeDtypeStruct((_ROWS, _NUM_PQ * _NUM_CODES), jnp.float32),
            jax.ShapeDtypeStruct((_NUM_PQ, _N_RB, 1, 1), jnp.float32),
        ],
        compiler_params=pltpu.CompilerParams(
            dimension_semantics=("parallel", "parallel")),
    )(z_g, codebooks)

    z_q = jnp.transpose(
        zq_g.reshape(_NUM_PQ, B, H, W, _PQ_DIM), (1, 0, 4, 2, 3)
    ).reshape(B, C, H, W)
    vq_loss = jnp.sum(loss) * (1.25 / (_NUM_PQ * _ROWS * _PQ_DIM))
    return z_q, vq_loss, prob
